# baseline (speedup 1.0000x reference)
import functools

import jax
import jax.numpy as jnp
import numpy as np
from jax.experimental import pallas as pl
from jax.experimental.pallas import tpu as pltpu
from jax.experimental.pallas import tpu_sc as plsc

SEQ = 32
NFREQ = 16
NSEG = 9
BATCH = 16
BT = 512
C = 512
HW = 256
D = 128
K = HW * D

KB = 4096
NSTEP = K // KB


def _build_dft() -> np.ndarray:
    t = np.arange(SEQ)
    f = np.arange(1, NFREQ + 1)
    ang = 2.0 * np.pi * np.outer(f, t) / SEQ
    cos = np.cos(ang) / np.sqrt(SEQ)
    msin = -np.sin(ang) / np.sqrt(SEQ)
    eye = np.eye(BATCH)
    fr = np.kron(eye, cos)
    fi = np.kron(eye, msin)
    return np.concatenate([fr, fi], axis=0).astype(np.float32)


_FCOMB = _build_dft()


def _tc_body(x_ref, w_ref, f_ref, amp_ref, acc_ref):
    j = pl.program_id(0)
    part = jax.lax.dot_general(
        x_ref[...], w_ref[...], (((1,), (1,)), ((), ())),
        preferred_element_type=jnp.float32)

    @pl.when(j == 0)
    def _():
        acc_ref[...] = part

    @pl.when(j > 0)
    def _():
        acc_ref[...] += part

    @pl.when(j == NSTEP - 1)
    def _():
        res = jax.lax.dot_general(
            f_ref[...], acc_ref[...], (((1,), (0,)), ((), ())),
            preferred_element_type=jnp.float32)
        re = res[:BATCH * NFREQ, :]
        im = res[BATCH * NFREQ:, :]
        mag = jnp.sqrt(re * re + im * im)
        amp_ref[...] = jnp.mean(mag, axis=1, keepdims=True)


def _tc_amp(xt2, w2, fcomb, interpret=False):
    return pl.pallas_call(
        _tc_body,
        grid=(NSTEP,),
        in_specs=[
            pl.BlockSpec((BT, KB), lambda j: (0, j)),
            pl.BlockSpec((C, KB), lambda j: (0, j)),
            pl.BlockSpec((BATCH * NFREQ * 2, BT), lambda j: (0, 0)),
        ],
        out_specs=pl.BlockSpec((BATCH * NFREQ, 1), lambda j: (0, 0)),
        out_shape=jax.ShapeDtypeStruct((BATCH * NFREQ, 1), jnp.float32),
        scratch_shapes=[pltpu.VMEM((BT, C), jnp.float32)],
        interpret=interpret,
    )(xt2, w2, fcomb)


def _sc_gate_body(amp_hbm, wgt_hbm, gates_hbm, load_hbm,
                  amp_v, wgt_v, gates_v, load_v):
    cid = jax.lax.axis_index("c")
    sid = jax.lax.axis_index("s")

    @pl.when(jnp.logical_and(cid == 0, sid == 0))
    def _():
        pltpu.sync_copy(amp_hbm, amp_v)
        pltpu.sync_copy(wgt_hbm, wgt_v)
        iota = jax.lax.iota(jnp.int32, 16)

        def shuf(v, sh):
            return v.at[iota ^ sh].get(mode="promise_in_bounds")

        def lane_sum(v):
            for sh in (8, 4, 2, 1):
                v = v + shuf(v, sh)
            return v

        def lane_max(v):
            for sh in (8, 4, 2, 1):
                v = jnp.maximum(v, shuf(v, sh))
            return v

        def lane_min(v):
            for sh in (8, 4, 2, 1):
                v = jnp.minimum(v, shuf(v, sh))
            return v

        ninf = jnp.float32(-3e38)
        pad = jnp.where(iota < NSEG, jnp.float32(0.0), ninf)
        counts = jnp.where(iota < 0, 1, 0)
        for b in range(BATCH):
            ab = amp_v[pl.ds(b * NFREQ, 16)]
            lvec = pad
            for s in range(NSEG):
                ls = lane_sum(ab * wgt_v[s, :])
                lvec = jnp.where(iota == s, ls, lvec)
            m1 = lane_max(lvec)
            j1 = lane_min(jnp.where(lvec == m1, iota, jnp.int32(16)))
            sel1 = iota == j1
            lvec2 = jnp.where(sel1, ninf, lvec)
            m2 = lane_max(lvec2)
            j2 = lane_min(jnp.where(lvec2 == m2, iota, jnp.int32(16)))
            sel2 = iota == j2
            t = jnp.exp(m2 - m1)
            g1 = 1.0 / (1.0 + t)
            g2 = t / (1.0 + t)
            gates_v[b, :] = jnp.where(sel1, g1, 0.0) + jnp.where(sel2, g2, 0.0)
            counts = counts + jnp.where(sel1, 1, 0) + jnp.where(sel2, 1, 0)
        load_v[...] = counts
        pltpu.sync_copy(gates_v, gates_hbm)
        pltpu.sync_copy(load_v, load_hbm)


@functools.cache
def _sc_gate():
    return pl.kernel(
        _sc_gate_body,
        mesh=plsc.VectorSubcoreMesh(core_axis_name="c", subcore_axis_name="s"),
        out_type=[
            jax.ShapeDtypeStruct((BATCH, 16), jnp.float32),
            jax.ShapeDtypeStruct((16,), jnp.int32),
        ],
        scratch_types=[
            pltpu.VMEM((BATCH * NFREQ,), jnp.float32),
            pltpu.VMEM((16, 16), jnp.float32),
            pltpu.VMEM((BATCH, 16), jnp.float32),
            pltpu.VMEM((16,), jnp.int32),
        ],
    )


def kernel(x, W_lin, b_lin, w_gate, training):
    del b_lin, training
    xt2 = jnp.transpose(x, (0, 1, 4, 2, 3)).reshape(BT, K)
    amp = _tc_amp(xt2, W_lin, jnp.asarray(_FCOMB)).reshape(BATCH * NFREQ)
    wgt_pad = jnp.zeros((16, 16), jnp.float32).at[:NSEG, :].set(w_gate.T)
    gates_pad, load_pad = _sc_gate()(amp, wgt_pad)
    return gates_pad[:, :NSEG], load_pad[:NSEG]


# speedup vs baseline: 4.1772x; 4.1772x over previous
import functools

import jax
import jax.numpy as jnp
import numpy as np
from jax.experimental import pallas as pl
from jax.experimental.pallas import tpu as pltpu
from jax.experimental.pallas import tpu_sc as plsc

SEQ = 32
NFREQ = 16
NSEG = 9
BATCH = 16
BT = 512
C = 512
HW = 256
D = 128
K = HW * D

DBLK = 8
NSTEP = D // DBLK
BTH = BT // 4


def _build_dft() -> np.ndarray:
    t = np.arange(SEQ)
    f = np.arange(1, NFREQ + 1)
    ang = 2.0 * np.pi * np.outer(f, t) / SEQ
    cos = np.cos(ang) / np.sqrt(SEQ)
    msin = -np.sin(ang) / np.sqrt(SEQ)
    eye = np.eye(BATCH)
    fr = np.kron(eye, cos)
    fi = np.kron(eye, msin)
    return np.concatenate([fr, fi], axis=0).astype(np.float32)


_FCOMB = _build_dft()


def _xt_body(x_ref, eye_ref, out_ref):
    m2 = x_ref[...].reshape(BTH * 128, D)
    q = jax.lax.dot_general(
        eye_ref[...], m2, (((1,), (1,)), ((), ())),
        preferred_element_type=jnp.float32)
    out_ref[...] = q.reshape(D, BTH, 128)


def _xt(x3, eye, interpret=False):
    return pl.pallas_call(
        _xt_body,
        grid=(4, 2),
        in_specs=[
            pl.BlockSpec((BTH, 128, D), lambda jb, jh: (jb, jh, 0)),
            pl.BlockSpec((128, 128), lambda jb, jh: (0, 0)),
        ],
        out_specs=pl.BlockSpec((D, BTH, 128), lambda jb, jh: (0, jb, jh)),
        out_shape=jax.ShapeDtypeStruct((D, BT, HW), jnp.float32),
        interpret=interpret,
    )(x3, eye)


def _tc_body(xt_ref, w_ref, f_ref, amp_ref, acc_ref):
    j = pl.program_id(0)
    part = jax.lax.dot_general(
        xt_ref[0], w_ref[:, pl.ds(0, HW)], (((1,), (1,)), ((), ())),
        preferred_element_type=jnp.float32)
    for dd in range(1, DBLK):
        part += jax.lax.dot_general(
            xt_ref[dd], w_ref[:, pl.ds(dd * HW, HW)], (((1,), (1,)), ((), ())),
            preferred_element_type=jnp.float32)

    @pl.when(j == 0)
    def _():
        acc_ref[...] = part

    @pl.when(j > 0)
    def _():
        acc_ref[...] += part

    @pl.when(j == NSTEP - 1)
    def _():
        res = jax.lax.dot_general(
            f_ref[...], acc_ref[...], (((1,), (0,)), ((), ())),
            preferred_element_type=jnp.float32)
        re = res[:BATCH * NFREQ, :]
        im = res[BATCH * NFREQ:, :]
        mag = jnp.sqrt(re * re + im * im)
        amp_ref[...] = jnp.mean(mag, axis=1, keepdims=True)


def _tc_amp(xt3, w2, fcomb, interpret=False):
    return pl.pallas_call(
        _tc_body,
        grid=(NSTEP,),
        in_specs=[
            pl.BlockSpec((DBLK, BT, HW), lambda j: (j, 0, 0)),
            pl.BlockSpec((C, DBLK * HW), lambda j: (0, j)),
            pl.BlockSpec((BATCH * NFREQ * 2, BT), lambda j: (0, 0)),
        ],
        out_specs=pl.BlockSpec((BATCH * NFREQ, 1), lambda j: (0, 0)),
        out_shape=jax.ShapeDtypeStruct((BATCH * NFREQ, 1), jnp.float32),
        scratch_shapes=[pltpu.VMEM((BT, C), jnp.float32)],
        interpret=interpret,
    )(xt3, w2, fcomb)


def _sc_gate_body(amp_hbm, wgt_hbm, gates_hbm, load_hbm,
                  amp_v, wgt_v, gates_v, load_v):
    cid = jax.lax.axis_index("c")
    sid = jax.lax.axis_index("s")

    @pl.when(jnp.logical_and(cid == 0, sid == 0))
    def _():
        pltpu.sync_copy(amp_hbm, amp_v)
        pltpu.sync_copy(wgt_hbm, wgt_v)
        iota = jax.lax.iota(jnp.int32, 16)

        def shuf(v, sh):
            return v.at[iota ^ sh].get(mode="promise_in_bounds")

        def lane_sum(v):
            for sh in (8, 4, 2, 1):
                v = v + shuf(v, sh)
            return v

        def lane_max(v):
            for sh in (8, 4, 2, 1):
                v = jnp.maximum(v, shuf(v, sh))
            return v

        def lane_min(v):
            for sh in (8, 4, 2, 1):
                v = jnp.minimum(v, shuf(v, sh))
            return v

        ninf = jnp.float32(-3e38)
        pad = jnp.where(iota < NSEG, jnp.float32(0.0), ninf)
        counts = jnp.where(iota < 0, 1, 0)
        for b in range(BATCH):
            ab = amp_v[pl.ds(b * NFREQ, 16)]
            lvec = pad
            for s in range(NSEG):
                ls = lane_sum(ab * wgt_v[s, :])
                lvec = jnp.where(iota == s, ls, lvec)
            m1 = lane_max(lvec)
            j1 = lane_min(jnp.where(lvec == m1, iota, jnp.int32(16)))
            sel1 = iota == j1
            lvec2 = jnp.where(sel1, ninf, lvec)
            m2 = lane_max(lvec2)
            j2 = lane_min(jnp.where(lvec2 == m2, iota, jnp.int32(16)))
            sel2 = iota == j2
            t = jnp.exp(m2 - m1)
            g1 = 1.0 / (1.0 + t)
            g2 = t / (1.0 + t)
            gates_v[b, :] = jnp.where(sel1, g1, 0.0) + jnp.where(sel2, g2, 0.0)
            counts = counts + jnp.where(sel1, 1, 0) + jnp.where(sel2, 1, 0)
        load_v[...] = counts
        pltpu.sync_copy(gates_v, gates_hbm)
        pltpu.sync_copy(load_v, load_hbm)


@functools.cache
def _sc_gate():
    return pl.kernel(
        _sc_gate_body,
        mesh=plsc.VectorSubcoreMesh(core_axis_name="c", subcore_axis_name="s"),
        out_type=[
            jax.ShapeDtypeStruct((BATCH, 16), jnp.float32),
            jax.ShapeDtypeStruct((16,), jnp.int32),
        ],
        scratch_types=[
            pltpu.VMEM((BATCH * NFREQ,), jnp.float32),
            pltpu.VMEM((16, 16), jnp.float32),
            pltpu.VMEM((BATCH, 16), jnp.float32),
            pltpu.VMEM((16,), jnp.int32),
        ],
    )


def kernel(x, W_lin, b_lin, w_gate, training):
    del b_lin, training
    eye = jnp.asarray(np.eye(128, dtype=np.float32))
    xt3 = _xt(x.reshape(BT, HW, D), eye)
    amp = _tc_amp(xt3, W_lin, jnp.asarray(_FCOMB)).reshape(BATCH * NFREQ)
    wgt_pad = jnp.zeros((16, 16), jnp.float32).at[:NSEG, :].set(w_gate.T)
    gates_pad, load_pad = _sc_gate()(amp, wgt_pad)
    return gates_pad[:, :NSEG], load_pad[:NSEG]
